# baseline (speedup 1.0000x reference)
import jax
import jax.numpy as jnp
from jax.experimental import pallas as pl
from jax.experimental.pallas import tpu as pltpu

D = 128
BLOCK = 8192


def _row_sum(v, ones_col):
    return jax.lax.dot_general(
        v, ones_col, (((1,), (0,)), ((), ())),
        preferred_element_type=jnp.float32,
        precision=jax.lax.Precision.HIGHEST)


def _rq_block(scalars_ref, x_ref, pi_ref, out_ref):
    c0 = scalars_ref[0]
    step = scalars_ref[1]
    inv_step = scalars_ref[2]
    kmax = scalars_ref[3]

    x = x_ref[...]
    pi = pi_ref[...]
    ones_col = jnp.ones((D, 1), jnp.float32)

    ssq = _row_sum(x * x, ones_col)
    inv = jax.lax.rsqrt(ssq)
    xn = x * inv
    xr = jax.lax.dot_general(
        xn, pi, (((1,), (1,)), ((), ())),
        preferred_element_type=jnp.float32,
        precision=jax.lax.Precision.DEFAULT)
    t = (xr - c0) * inv_step
    idx = jnp.clip(jnp.round(t), 0.0, kmax)
    r = t - idx
    scale = _row_sum(jnp.abs(r), ones_col) * (step / D)
    sbits = jax.lax.bitcast_convert_type(r, jnp.int32) & (-2147483648)
    corr = jax.lax.bitcast_convert_type(
        jax.lax.bitcast_convert_type(
            jnp.broadcast_to(scale, r.shape), jnp.int32) | sbits,
        jnp.float32)
    xc = (c0 + idx * step) + corr
    out_rot = jax.lax.dot_general(
        xc, pi, (((1,), (0,)), ((), ())),
        preferred_element_type=jnp.float32,
        precision=jax.lax.Precision.DEFAULT)
    out_ref[...] = out_rot * (ssq * inv)


def kernel(x, Pi, centroids):
    n = x.shape[0]
    k = centroids.shape[0]
    c0 = centroids[0]
    step = centroids[1] - centroids[0]
    scalars = jnp.stack(
        [c0, step, 1.0 / step, jnp.float32(k - 1)]).astype(jnp.float32)
    grid = (n // BLOCK,)
    return pl.pallas_call(
        _rq_block,
        grid=grid,
        in_specs=[
            pl.BlockSpec(memory_space=pltpu.SMEM),
            pl.BlockSpec((BLOCK, D), lambda i: (i, 0)),
            pl.BlockSpec((D, D), lambda i: (0, 0)),
        ],
        out_specs=pl.BlockSpec((BLOCK, D), lambda i: (i, 0)),
        out_shape=jax.ShapeDtypeStruct((n, D), jnp.float32),
        compiler_params=pltpu.CompilerParams(
            dimension_semantics=("parallel",)),
    )(scalars, x, Pi)


# speedup vs baseline: 3.6129x; 3.6129x over previous
import jax
import jax.numpy as jnp
from jax.experimental import pallas as pl
from jax.experimental.pallas import tpu as pltpu

D = 128
BLOCK = 8192


def _rq_block(scalars_ref, x_ref, pi_ref, out_ref):
    c0 = scalars_ref[0]
    step = scalars_ref[1]
    inv_step = scalars_ref[2]
    kmax = scalars_ref[3]

    x = x_ref[...]
    pi = pi_ref[...]
    ssq = jnp.sum(x * x, axis=1, keepdims=True)
    inv = jax.lax.rsqrt(ssq)
    xn = x * inv
    xr = jax.lax.dot_general(
        xn, pi, (((1,), (1,)), ((), ())),
        preferred_element_type=jnp.float32,
        precision=jax.lax.Precision.DEFAULT)
    t = (xr - c0) * inv_step
    idx = jnp.clip(jnp.round(t), 0.0, kmax)
    r = t - idx
    scale = jnp.sum(jnp.abs(r), axis=1, keepdims=True) * (step / D)
    sbits = jax.lax.bitcast_convert_type(r, jnp.int32) & (-2147483648)
    corr = jax.lax.bitcast_convert_type(
        jax.lax.bitcast_convert_type(
            jnp.broadcast_to(scale, r.shape), jnp.int32) | sbits,
        jnp.float32)
    xc = (c0 + idx * step) + corr
    out_rot = jax.lax.dot_general(
        xc, pi, (((1,), (0,)), ((), ())),
        preferred_element_type=jnp.float32,
        precision=jax.lax.Precision.DEFAULT)
    out_ref[...] = out_rot * (ssq * inv)


def kernel(x, Pi, centroids):
    n = x.shape[0]
    k = centroids.shape[0]
    c0 = centroids[0]
    step = centroids[1] - centroids[0]
    scalars = jnp.stack(
        [c0, step, 1.0 / step, jnp.float32(k - 1)]).astype(jnp.float32)
    grid = (n // BLOCK,)
    return pl.pallas_call(
        _rq_block,
        grid=grid,
        in_specs=[
            pl.BlockSpec(memory_space=pltpu.SMEM),
            pl.BlockSpec((BLOCK, D), lambda i: (i, 0)),
            pl.BlockSpec((D, D), lambda i: (0, 0)),
        ],
        out_specs=pl.BlockSpec((BLOCK, D), lambda i: (i, 0)),
        out_shape=jax.ShapeDtypeStruct((n, D), jnp.float32),
        compiler_params=pltpu.CompilerParams(
            dimension_semantics=("parallel",)),
    )(scalars, x, Pi)


# speedup vs baseline: 3.6715x; 1.0162x over previous
import jax
import jax.numpy as jnp
from jax.experimental import pallas as pl
from jax.experimental.pallas import tpu as pltpu

D = 128
BLOCK = 16384


def _rq_block(scalars_ref, x_ref, pi_ref, out_ref):
    c0 = scalars_ref[0]
    step = scalars_ref[1]
    inv_step = scalars_ref[2]
    kmax = scalars_ref[3]

    x = x_ref[...]
    pi = pi_ref[...]
    ssq = jnp.sum(x * x, axis=1, keepdims=True)
    inv = jax.lax.rsqrt(ssq)
    xn = x * inv
    xr = jax.lax.dot_general(
        xn, pi, (((1,), (1,)), ((), ())),
        preferred_element_type=jnp.float32,
        precision=jax.lax.Precision.DEFAULT)
    t = (xr - c0) * inv_step
    idx = jnp.clip(jnp.round(t), 0.0, kmax)
    r = t - idx
    scale = jnp.sum(jnp.abs(r), axis=1, keepdims=True) * (step / D)
    sbits = jax.lax.bitcast_convert_type(r, jnp.int32) & (-2147483648)
    corr = jax.lax.bitcast_convert_type(
        jax.lax.bitcast_convert_type(
            jnp.broadcast_to(scale, r.shape), jnp.int32) | sbits,
        jnp.float32)
    xc = (c0 + idx * step) + corr
    out_rot = jax.lax.dot_general(
        xc, pi, (((1,), (0,)), ((), ())),
        preferred_element_type=jnp.float32,
        precision=jax.lax.Precision.DEFAULT)
    out_ref[...] = out_rot * (ssq * inv)


def kernel(x, Pi, centroids):
    n = x.shape[0]
    k = centroids.shape[0]
    c0 = centroids[0]
    step = centroids[1] - centroids[0]
    scalars = jnp.stack(
        [c0, step, 1.0 / step, jnp.float32(k - 1)]).astype(jnp.float32)
    grid = (n // BLOCK,)
    return pl.pallas_call(
        _rq_block,
        grid=grid,
        in_specs=[
            pl.BlockSpec(memory_space=pltpu.SMEM),
            pl.BlockSpec((BLOCK, D), lambda i: (i, 0)),
            pl.BlockSpec((D, D), lambda i: (0, 0)),
        ],
        out_specs=pl.BlockSpec((BLOCK, D), lambda i: (i, 0)),
        out_shape=jax.ShapeDtypeStruct((n, D), jnp.float32),
        compiler_params=pltpu.CompilerParams(
            dimension_semantics=("parallel",)),
    )(scalars, x, Pi)
